# Initial kernel scaffold; baseline (speedup 1.0000x reference)
import jax
import jax.numpy as jnp
from jax.experimental import pallas as pl


def kernel(word, glove_table, fast_text_table, Wg, bg, Wf, bf):
    raise NotImplementedError("write your pallas kernel here")


# speedup vs baseline: 30.7510x; 30.7510x over previous
import functools

import jax
import jax.numpy as jnp
from jax import lax
from jax.experimental import pallas as pl
from jax.experimental.pallas import tpu as pltpu
from jax.experimental.pallas import tpu_sc as plsc

V = 1000
NC, NS, L = 2, 16, 16
NW = NC * NS
N_TOK = 4096 * 20
PER_W = N_TOK // NW
STEPS = PER_W // L


def _stage1_body(glove_ref, fast_ref, wg_ref, bg_ref, wf_ref, bf_ref, s_ref):
    sg = jnp.sum(wg_ref[...], axis=1)
    sf = jnp.sum(wf_ref[...], axis=1)
    c = jnp.sum(bg_ref[...]) + jnp.sum(bf_ref[...])
    s = (jnp.sum(glove_ref[...] * sg[None, :], axis=1)
         + jnp.sum(fast_ref[...] * sf[None, :], axis=1) + c)
    s_ref[...] = s


def _build_s_table(glove, fast, Wg, bg, Wf, bf):
    return pl.pallas_call(
        _stage1_body,
        out_shape=jax.ShapeDtypeStruct((V,), jnp.float32),
    )(glove, fast, Wg, bg, Wf, bf)


@functools.lru_cache(maxsize=1)
def _make_gather_sum():
    @functools.partial(
        pl.kernel,
        out_type=jax.ShapeDtypeStruct((NW, L), jnp.float32),
        mesh=plsc.VectorSubcoreMesh(
            core_axis_name="c", subcore_axis_name="s",
            num_cores=NC, num_subcores=NS),
        scratch_types=[
            pltpu.VMEM((PER_W,), jnp.int32),
            pltpu.VMEM((V,), jnp.float32),
            pltpu.VMEM((L,), jnp.float32),
        ],
        compiler_params=pltpu.CompilerParams(needs_layout_passes=False),
    )
    def _gather_sum(word_hbm, s_hbm, out_hbm, idx_v, tab_v, acc_v):
        wid = lax.axis_index("s") * NC + lax.axis_index("c")
        base = wid * PER_W
        pltpu.sync_copy(word_hbm.at[pl.ds(base, PER_W)], idx_v)
        pltpu.sync_copy(s_hbm, tab_v)

        def step(i, acc):
            iv = idx_v[pl.ds(i * L, L)]
            return acc + plsc.load_gather(tab_v, [iv])

        acc_v[...] = lax.fori_loop(0, STEPS, step,
                                   jnp.zeros((L,), jnp.float32))
        pltpu.sync_copy(acc_v, out_hbm.at[wid])

    return _gather_sum


def kernel(word, glove_table, fast_text_table, Wg, bg, Wf, bf):
    s_tab = _build_s_table(glove_table, fast_text_table, Wg, bg, Wf, bf)
    word_flat = word.reshape(-1).astype(jnp.int32)
    partials = _make_gather_sum()(word_flat, s_tab)
    return jnp.sum(partials)


# speedup vs baseline: 31.9187x; 1.0380x over previous
import functools

import jax
import jax.numpy as jnp
from jax import lax
from jax.experimental import pallas as pl
from jax.experimental.pallas import tpu as pltpu
from jax.experimental.pallas import tpu_sc as plsc

V = 1000
NC, NS, L = 2, 16, 16
NW = NC * NS
N_TOK = 4096 * 20
PER_W = N_TOK // NW
STEPS = PER_W // L


def _stage1_body(glove_ref, fast_ref, wg_ref, bg_ref, wf_ref, bf_ref, s_ref):
    sg = jnp.sum(wg_ref[...], axis=1)
    sf = jnp.sum(wf_ref[...], axis=1)
    c = jnp.sum(bg_ref[...]) + jnp.sum(bf_ref[...])
    s = (jnp.sum(glove_ref[...] * sg[None, :], axis=1)
         + jnp.sum(fast_ref[...] * sf[None, :], axis=1) + c)
    s_ref[...] = s


def _build_s_table(glove, fast, Wg, bg, Wf, bf):
    return pl.pallas_call(
        _stage1_body,
        out_shape=jax.ShapeDtypeStruct((V,), jnp.float32),
    )(glove, fast, Wg, bg, Wf, bf)


_UNROLL = 8


@functools.lru_cache(maxsize=1)
def _make_gather_sum():
    @functools.partial(
        pl.kernel,
        out_type=jax.ShapeDtypeStruct((NW, L), jnp.float32),
        mesh=plsc.VectorSubcoreMesh(
            core_axis_name="c", subcore_axis_name="s",
            num_cores=NC, num_subcores=NS),
        scratch_types=[
            pltpu.VMEM((PER_W,), jnp.int32),
            pltpu.VMEM((V,), jnp.float32),
            pltpu.VMEM((L,), jnp.float32),
            pltpu.SemaphoreType.DMA,
            pltpu.SemaphoreType.DMA,
        ],
        compiler_params=pltpu.CompilerParams(needs_layout_passes=False),
    )
    def _gather_sum(word_hbm, s_hbm, out_hbm, idx_v, tab_v, acc_v,
                    sem_i, sem_t):
        wid = lax.axis_index("s") * NC + lax.axis_index("c")
        base = wid * PER_W
        cp_i = pltpu.async_copy(word_hbm.at[pl.ds(base, PER_W)], idx_v, sem_i)
        cp_t = pltpu.async_copy(s_hbm, tab_v, sem_t)
        cp_t.wait()
        cp_i.wait()

        def step(i, accs):
            out = list(accs)
            for j in range(_UNROLL):
                iv = idx_v[pl.ds((i * _UNROLL + j) * L, L)]
                out[j % 4] = out[j % 4] + plsc.load_gather(tab_v, [iv])
            return tuple(out)

        zeros = jnp.zeros((L,), jnp.float32)
        accs = lax.fori_loop(0, STEPS // _UNROLL, step,
                             (zeros, zeros, zeros, zeros))
        acc_v[...] = (accs[0] + accs[1]) + (accs[2] + accs[3])
        pltpu.sync_copy(acc_v, out_hbm.at[wid])

    return _gather_sum


def kernel(word, glove_table, fast_text_table, Wg, bg, Wf, bf):
    s_tab = _build_s_table(glove_table, fast_text_table, Wg, bg, Wf, bf)
    word_flat = word.reshape(-1).astype(jnp.int32)
    partials = _make_gather_sum()(word_flat, s_tab)
    return jnp.sum(partials)


# speedup vs baseline: 34.1031x; 1.0684x over previous
import functools

import jax
import jax.numpy as jnp
from jax import lax
from jax.experimental import pallas as pl
from jax.experimental.pallas import tpu as pltpu
from jax.experimental.pallas import tpu_sc as plsc

V = 1000
NC, NS, L = 2, 16, 16
NW = NC * NS
N_TOK = 4096 * 20
PER_W = N_TOK // NW
STEPS = PER_W // L


def _stage1_body(glove_ref, fast_ref, wg_ref, bg_ref, wf_ref, bf_ref, s_ref):
    sg = jnp.sum(wg_ref[...], axis=1)
    sf = jnp.sum(wf_ref[...], axis=1)
    c = jnp.sum(bg_ref[...]) + jnp.sum(bf_ref[...])
    s = (jnp.sum(glove_ref[...] * sg[None, :], axis=1)
         + jnp.sum(fast_ref[...] * sf[None, :], axis=1) + c)
    s_ref[...] = s


def _build_s_table(glove, fast, Wg, bg, Wf, bf):
    return pl.pallas_call(
        _stage1_body,
        out_shape=jax.ShapeDtypeStruct((V,), jnp.float32),
    )(glove, fast, Wg, bg, Wf, bf)


_UNROLL = 8


@functools.lru_cache(maxsize=1)
def _make_gather_sum():
    @functools.partial(
        pl.kernel,
        out_type=jax.ShapeDtypeStruct((NW, L), jnp.float32),
        mesh=plsc.VectorSubcoreMesh(
            core_axis_name="c", subcore_axis_name="s",
            num_cores=NC, num_subcores=NS),
        scratch_types=[
            pltpu.VMEM((PER_W,), jnp.int32),
            pltpu.VMEM((V,), jnp.float32),
            pltpu.VMEM((L,), jnp.float32),
            pltpu.SemaphoreType.DMA,
            pltpu.SemaphoreType.DMA,
        ],
        compiler_params=pltpu.CompilerParams(needs_layout_passes=False),
    )
    def _gather_sum(word_hbm, s_hbm, out_hbm, idx_v, tab_v, acc_v,
                    sem_i, sem_t):
        wid = lax.axis_index("s") * NC + lax.axis_index("c")
        base = wid * PER_W
        cp_i = pltpu.async_copy(word_hbm.at[pl.ds(base, PER_W)], idx_v, sem_i)
        cp_t = pltpu.async_copy(s_hbm, tab_v, sem_t)
        cp_t.wait()
        cp_i.wait()

        def step(i, accs):
            out = list(accs)
            for j in range(_UNROLL):
                iv = idx_v[pl.ds((i * _UNROLL + j) * L, L)]
                out[j % 4] = out[j % 4] + plsc.load_gather(tab_v, [iv])
            return tuple(out)

        zeros = jnp.zeros((L,), jnp.float32)
        accs = lax.fori_loop(0, STEPS // _UNROLL, step,
                             (zeros, zeros, zeros, zeros))
        acc_v[...] = (accs[0] + accs[1]) + (accs[2] + accs[3])
        pltpu.sync_copy(acc_v, out_hbm.at[wid])

    return _gather_sum


V_PAD = 1008


@functools.lru_cache(maxsize=1)
def _make_histogram():
    @functools.partial(
        pl.kernel,
        out_type=jax.ShapeDtypeStruct((NW, V), jnp.float32),
        mesh=plsc.VectorSubcoreMesh(
            core_axis_name="c", subcore_axis_name="s",
            num_cores=NC, num_subcores=NS),
        scratch_types=[
            pltpu.VMEM((PER_W,), jnp.int32),
            pltpu.VMEM((V_PAD,), jnp.float32),
            pltpu.SemaphoreType.DMA,
        ],
        compiler_params=pltpu.CompilerParams(needs_layout_passes=False,
                                             use_tc_tiling_on_sc=False),
    )
    def _histogram(word_hbm, out_hbm, idx_v, hist_v, sem_i):
        wid = lax.axis_index("s") * NC + lax.axis_index("c")
        base = wid * PER_W
        cp_i = pltpu.async_copy(word_hbm.at[pl.ds(base, PER_W)], idx_v, sem_i)
        zeros = jnp.zeros((L,), jnp.float32)

        def zstep(k, _):
            hist_v[pl.ds(k * L, L)] = zeros
            return 0

        lax.fori_loop(0, V_PAD // L, zstep, 0)
        cp_i.wait()

        def step(i, _):
            for j in range(_UNROLL):
                iv = idx_v[pl.ds((i * _UNROLL + j) * L, L)]
                cnt, last = plsc.scan_count(iv)
                plsc.addupdate_scatter(hist_v, [iv],
                                       cnt.astype(jnp.float32), mask=last)
            return 0

        lax.fori_loop(0, STEPS // _UNROLL, step, 0)
        pltpu.sync_copy(hist_v.at[pl.ds(0, V)], out_hbm.at[wid])

    return _histogram


def _final_body(hists_ref, glove_ref, fast_ref, wg_ref, bg_ref, wf_ref,
                bf_ref, out_ref):
    counts = jnp.sum(hists_ref[...], axis=0)
    sg = jnp.sum(wg_ref[...], axis=1)
    sf = jnp.sum(wf_ref[...], axis=1)
    s = (jnp.sum(glove_ref[...] * sg[None, :], axis=1)
         + jnp.sum(fast_ref[...] * sf[None, :], axis=1))
    c = jnp.sum(bg_ref[...]) + jnp.sum(bf_ref[...])
    out_ref[0, 0] = jnp.sum(counts * s) + N_TOK * c


def kernel(word, glove_table, fast_text_table, Wg, bg, Wf, bf):
    word_flat = word.reshape(-1).astype(jnp.int32)
    hists = _make_histogram()(word_flat)
    total = pl.pallas_call(
        _final_body,
        out_shape=jax.ShapeDtypeStruct((1, 1), jnp.float32),
        out_specs=pl.BlockSpec(memory_space=pltpu.SMEM),
    )(hists, glove_table, fast_text_table, Wg, bg, Wf, bf)
    return jnp.reshape(total, ())
